# Initial kernel scaffold; baseline (speedup 1.0000x reference)
import jax
import jax.numpy as jnp
from jax.experimental import pallas as pl


def kernel(context_words, target_word, negative_samples, emb_weight, ctx_weight):
    raise NotImplementedError("write your pallas kernel here")


# speedup vs baseline: 4.7893x; 4.7893x over previous
import dataclasses
import functools

import jax
import jax.numpy as jnp
from jax import lax
from jax.experimental import pallas as pl
from jax.experimental.pallas import tpu as pltpu
from jax.experimental.pallas import tpu_sc as plsc

V = 1000000
D = 64
B = 16384
C = 20
NNEG = 20

NC = 2
NS = 16
NW = NC * NS
BPW = B // NW
BK = 32
NCHUNK = BPW // BK
ROWS = BK * C
NSUB = ROWS // 128


def _sc_body(emb_hbm, ctxw_hbm, ctx_idx_hbm, tgt_idx_hbm, neg_idx_hbm,
             pos_hbm, negs_hbm,
             ctx_idx_v, neg_idx_v, tgt_idx_v,
             ctx_rows_v, neg_rows_v, tgt_rows_v,
             pos_buf, neg_buf, sem):
    wid = lax.axis_index("s") * NC + lax.axis_index("c")

    pltpu.sync_copy(ctx_idx_hbm.at[pl.ds(wid * (ROWS // 128 * NCHUNK),
                                         ROWS // 128 * NCHUNK)], ctx_idx_v)
    pltpu.sync_copy(neg_idx_hbm.at[pl.ds(wid * (ROWS // 128 * NCHUNK),
                                         ROWS // 128 * NCHUNK)], neg_idx_v)
    pltpu.sync_copy(tgt_idx_hbm.at[wid], tgt_idx_v)

    lanes = lax.iota(jnp.int32, 16)

    @pl.loop(0, NCHUNK)
    def _(c):
        copies = []
        for j in range(NSUB):
            copies.append(pltpu.async_copy(
                emb_hbm.at[ctx_idx_v.at[c * NSUB + j]],
                ctx_rows_v.at[pl.ds(j * 128, 128)], sem))
        for j in range(NSUB):
            copies.append(pltpu.async_copy(
                ctxw_hbm.at[neg_idx_v.at[c * NSUB + j]],
                neg_rows_v.at[pl.ds(j * 128, 128)], sem))
        copies.append(pltpu.async_copy(
            ctxw_hbm.at[tgt_idx_v.at[c]], tgt_rows_v, sem))
        for cp in copies:
            cp.wait()

        @pl.loop(0, BK)
        def _(b):
            m = [ctx_rows_v[b * C, pl.ds(k * 16, 16)] for k in range(4)]
            for i in range(1, C):
                for k in range(4):
                    m[k] = m[k] + ctx_rows_v[b * C + i, pl.ds(k * 16, 16)]
            acc = m[0] * tgt_rows_v[b, pl.ds(0, 16)]
            for k in range(1, 4):
                acc = acc + m[k] * tgt_rows_v[b, pl.ds(k * 16, 16)]
            s = jnp.sum(acc)
            row = b // 16
            lane = b % 16
            pos_buf[row, :] = jnp.where(lanes == lane, s, pos_buf[row, :])
            for n in range(NNEG):
                r = b * NNEG + n
                acc = m[0] * neg_rows_v[r, pl.ds(0, 16)]
                for k in range(1, 4):
                    acc = acc + m[k] * neg_rows_v[r, pl.ds(k * 16, 16)]
                s = jnp.sum(acc)
                nrow = r // 16
                nlane = r % 16
                neg_buf[nrow, :] = jnp.where(lanes == nlane, s,
                                             neg_buf[nrow, :])

        pltpu.sync_copy(pos_buf, pos_hbm.at[pl.ds(wid * (BPW // 16) + c * (BK // 16),
                                                  BK // 16)])
        pltpu.sync_copy(neg_buf,
                        negs_hbm.at[pl.ds(wid * (BPW * NNEG // 16) + c * (ROWS // 16),
                                          ROWS // 16)])


_sc_cp = pltpu.CompilerParams()
if "needs_layout_passes" in pltpu.CompilerParams.__dataclass_fields__:
    _sc_cp = dataclasses.replace(_sc_cp, needs_layout_passes=False)
if "use_tc_tiling_on_sc" in pltpu.CompilerParams.__dataclass_fields__:
    _sc_cp = dataclasses.replace(_sc_cp, use_tc_tiling_on_sc=False)

_sc_scores = functools.partial(
    pl.kernel,
    compiler_params=_sc_cp,
    out_type=(jax.ShapeDtypeStruct((B // 16, 16), jnp.float32),
              jax.ShapeDtypeStruct((B * NNEG // 16, 16), jnp.float32)),
    mesh=plsc.VectorSubcoreMesh(core_axis_name="c", subcore_axis_name="s"),
    scratch_types=[
        pltpu.VMEM((NSUB * NCHUNK, 128), jnp.int32),
        pltpu.VMEM((NSUB * NCHUNK, 128), jnp.int32),
        pltpu.VMEM((NCHUNK, BK), jnp.int32),
        pltpu.VMEM((ROWS, D), jnp.float32),
        pltpu.VMEM((ROWS, D), jnp.float32),
        pltpu.VMEM((BK, D), jnp.float32),
        pltpu.VMEM((BK // 16, 16), jnp.float32),
        pltpu.VMEM((ROWS // 16, 16), jnp.float32),
        pltpu.SemaphoreType.DMA,
    ],
)(_sc_body)


def _loss_body(pos_ref, neg_ref, o_ref):
    inv_c = jnp.float32(1.0 / C)

    def ls(x):
        return jnp.minimum(x, 0.0) - jnp.log1p(jnp.exp(-jnp.abs(x)))

    pos = pos_ref[...] * inv_c
    neg = neg_ref[...] * inv_c
    total = jnp.sum(ls(pos)) + jnp.sum(ls(-neg))
    o_ref[0, 0] = -(total / jnp.float32(B))


_loss = pl.pallas_call(
    _loss_body,
    out_shape=jax.ShapeDtypeStruct((1, 1), jnp.float32),
    out_specs=pl.BlockSpec(memory_space=pltpu.SMEM),
)


def kernel(context_words, target_word, negative_samples, emb_weight, ctx_weight):
    ctx_idx = context_words.astype(jnp.int32).reshape(B * C // 128, 128)
    neg_idx = negative_samples.astype(jnp.int32).reshape(B * NNEG // 128, 128)
    tgt_idx = target_word.astype(jnp.int32).reshape(NW, NCHUNK, BK)
    pos_raw, neg_raw = _sc_scores(emb_weight, ctx_weight, ctx_idx, tgt_idx,
                                  neg_idx)
    loss = _loss(pos_raw.reshape(128, 128), neg_raw.reshape(2560, 128))
    return loss[0, 0]


# speedup vs baseline: 5.0083x; 1.0457x over previous
import dataclasses
import functools

import jax
import jax.numpy as jnp
from jax import lax
from jax.experimental import pallas as pl
from jax.experimental.pallas import tpu as pltpu
from jax.experimental.pallas import tpu_sc as plsc

V = 1000000
D = 64
B = 16384
C = 20
NNEG = 20

NC = 2
NS = 16
NW = NC * NS
BPW = B // NW
BK = 16
NCHUNK = BPW // BK
ROWS = BK * C
SUB = 64
NSUB = ROWS // SUB


def _sc_body(emb_hbm, ctxw_hbm, ctx_idx_hbm, tgt_idx_hbm, neg_idx_hbm,
             pos_hbm, negs_hbm,
             ctx_idx_v, neg_idx_v, tgt_idx_v,
             ctx_rows0, neg_rows0, tgt_rows0,
             ctx_rows1, neg_rows1, tgt_rows1,
             pos_acc, neg_acc, sem0, sem1):
    wid = lax.axis_index("s") * NC + lax.axis_index("c")

    pltpu.sync_copy(ctx_idx_hbm.at[pl.ds(wid * (BPW * C // SUB),
                                         BPW * C // SUB)], ctx_idx_v)
    pltpu.sync_copy(neg_idx_hbm.at[pl.ds(wid * (BPW * NNEG // SUB),
                                         BPW * NNEG // SUB)], neg_idx_v)
    pltpu.sync_copy(tgt_idx_hbm.at[wid], tgt_idx_v)

    lanes = lax.iota(jnp.int32, 16)
    bufs = ((ctx_rows0, neg_rows0, tgt_rows0, sem0),
            (ctx_rows1, neg_rows1, tgt_rows1, sem1))

    def fire(c, par):
        ctx_rows, neg_rows, tgt_rows, sem = bufs[par]
        for j in range(NSUB):
            pltpu.async_copy(emb_hbm.at[ctx_idx_v.at[c * NSUB + j]],
                             ctx_rows.at[pl.ds(j * SUB, SUB)], sem)
            pltpu.async_copy(ctxw_hbm.at[neg_idx_v.at[c * NSUB + j]],
                             neg_rows.at[pl.ds(j * SUB, SUB)], sem)
        pltpu.async_copy(ctxw_hbm.at[tgt_idx_v.at[c]], tgt_rows, sem)

    def drain(c, par):
        ctx_rows, neg_rows, tgt_rows, sem = bufs[par]
        for j in range(NSUB):
            pltpu.make_async_copy(emb_hbm.at[ctx_idx_v.at[c * NSUB + j]],
                                  ctx_rows.at[pl.ds(j * SUB, SUB)], sem).wait()
            pltpu.make_async_copy(ctxw_hbm.at[neg_idx_v.at[c * NSUB + j]],
                                  neg_rows.at[pl.ds(j * SUB, SUB)], sem).wait()
        pltpu.make_async_copy(ctxw_hbm.at[tgt_idx_v.at[c]], tgt_rows,
                              sem).wait()

    def compute(c, par):
        ctx_rows, neg_rows, tgt_rows, _ = bufs[par]

        @pl.loop(0, BK)
        def _(b):
            m = [ctx_rows[b * C, pl.ds(k * 16, 16)] for k in range(4)]
            for i in range(1, C):
                for k in range(4):
                    m[k] = m[k] + ctx_rows[b * C + i, pl.ds(k * 16, 16)]
            acc = m[0] * tgt_rows[b, pl.ds(0, 16)]
            for k in range(1, 4):
                acc = acc + m[k] * tgt_rows[b, pl.ds(k * 16, 16)]
            s = jnp.sum(acc)
            pos_acc[c, :] = jnp.where(lanes == b, s, pos_acc[c, :])
            for n in range(NNEG):
                r = b * NNEG + n
                acc = m[0] * neg_rows[r, pl.ds(0, 16)]
                for k in range(1, 4):
                    acc = acc + m[k] * neg_rows[r, pl.ds(k * 16, 16)]
                s = jnp.sum(acc)
                g = c * ROWS + r
                nrow = g // 16
                nlane = g % 16
                neg_acc[nrow, :] = jnp.where(lanes == nlane, s,
                                             neg_acc[nrow, :])

    fire(0, 0)

    @pl.loop(0, NCHUNK, step=2)
    def _(c):
        fire(c + 1, 1)
        drain(c, 0)
        compute(c, 0)

        @pl.when(c + 2 < NCHUNK)
        def _():
            fire(c + 2, 0)

        drain(c + 1, 1)
        compute(c + 1, 1)

    pltpu.sync_copy(pos_acc, pos_hbm.at[pl.ds(wid * (BPW // 16), BPW // 16)])
    pltpu.sync_copy(neg_acc,
                    negs_hbm.at[pl.ds(wid * (BPW * NNEG // 16),
                                      BPW * NNEG // 16)])


_sc_cp = pltpu.CompilerParams()
if "needs_layout_passes" in pltpu.CompilerParams.__dataclass_fields__:
    _sc_cp = dataclasses.replace(_sc_cp, needs_layout_passes=False)
if "use_tc_tiling_on_sc" in pltpu.CompilerParams.__dataclass_fields__:
    _sc_cp = dataclasses.replace(_sc_cp, use_tc_tiling_on_sc=False)

_sc_scores = functools.partial(
    pl.kernel,
    compiler_params=_sc_cp,
    out_type=(jax.ShapeDtypeStruct((B // 16, 16), jnp.float32),
              jax.ShapeDtypeStruct((B * NNEG // 16, 16), jnp.float32)),
    mesh=plsc.VectorSubcoreMesh(core_axis_name="c", subcore_axis_name="s"),
    scratch_types=[
        pltpu.VMEM((BPW * C // SUB, SUB), jnp.int32),
        pltpu.VMEM((BPW * NNEG // SUB, SUB), jnp.int32),
        pltpu.VMEM((NCHUNK, BK), jnp.int32),
        pltpu.VMEM((ROWS, D), jnp.float32),
        pltpu.VMEM((ROWS, D), jnp.float32),
        pltpu.VMEM((BK, D), jnp.float32),
        pltpu.VMEM((ROWS, D), jnp.float32),
        pltpu.VMEM((ROWS, D), jnp.float32),
        pltpu.VMEM((BK, D), jnp.float32),
        pltpu.VMEM((BPW // 16, 16), jnp.float32),
        pltpu.VMEM((BPW * NNEG // 16, 16), jnp.float32),
        pltpu.SemaphoreType.DMA,
        pltpu.SemaphoreType.DMA,
    ],
)(_sc_body)


def _loss_body(pos_ref, neg_ref, o_ref):
    inv_c = jnp.float32(1.0 / C)

    def ls(x):
        return jnp.minimum(x, 0.0) - jnp.log1p(jnp.exp(-jnp.abs(x)))

    pos = pos_ref[...] * inv_c
    neg = neg_ref[...] * inv_c
    total = jnp.sum(ls(pos)) + jnp.sum(ls(-neg))
    o_ref[0, 0] = -(total / jnp.float32(B))


_loss = pl.pallas_call(
    _loss_body,
    out_shape=jax.ShapeDtypeStruct((1, 1), jnp.float32),
    out_specs=pl.BlockSpec(memory_space=pltpu.SMEM),
)


def kernel(context_words, target_word, negative_samples, emb_weight, ctx_weight):
    ctx_idx = context_words.astype(jnp.int32).reshape(B * C // SUB, SUB)
    neg_idx = negative_samples.astype(jnp.int32).reshape(B * NNEG // SUB, SUB)
    tgt_idx = target_word.astype(jnp.int32).reshape(NW, NCHUNK, BK)
    pos_raw, neg_raw = _sc_scores(emb_weight, ctx_weight, ctx_idx, tgt_idx,
                                  neg_idx)
    loss = _loss(pos_raw.reshape(128, 128), neg_raw.reshape(2560, 128))
    return loss[0, 0]
